# baseline (speedup 1.0000x reference)
import functools

import jax
import jax.numpy as jnp
from jax import lax
from jax.experimental import pallas as pl
from jax.experimental.pallas import tpu as pltpu
from jax.experimental.pallas import tpu_sc as plsc

N_NODE = 10000
H = 64
F = 128
E = 320000
BATCH = 16384

NT = 16
NP = 10240
EPT = E // NT
K = 400
NCHUNK = EPT // K
NPT = NP // NT
NB = 64
BM = 2560
BPW = BATCH // 32

_f32 = jnp.float32
_i32 = jnp.int32


def _dense_in_body(x_ref, w_ref, a2_ref, u1_ref, sc_ref):
    x = x_ref[0]
    w = w_ref[0]
    u1 = jnp.dot(x, w, preferred_element_type=_f32)
    u1_ref[0] = u1
    a2 = a2_ref[0]
    sc_ref[0] = lax.dot_general(a2, u1, (((1,), (1,)), ((), ())),
                                preferred_element_type=_f32)


def _dense_in(xs, wt, att2):
    return pl.pallas_call(
        _dense_in_body,
        grid=(2, NP // BM),
        in_specs=[
            pl.BlockSpec((1, BM, F), lambda g, m: (g, m, 0)),
            pl.BlockSpec((1, F, H), lambda g, m: (g, 0, 0)),
            pl.BlockSpec((1, 2, H), lambda g, m: (g, 0, 0)),
        ],
        out_specs=[
            pl.BlockSpec((1, BM, H), lambda g, m: (g, m, 0)),
            pl.BlockSpec((1, 2, BM), lambda g, m: (g, 0, m)),
        ],
        out_shape=[
            jax.ShapeDtypeStruct((2, NP, H), _f32),
            jax.ShapeDtypeStruct((2, 2, NP), _f32),
        ],
    )(xs, wt, att2)


def _agg_body(feat, scores, srcs, dsts, etype, gw, out,
              srcr0, srcr1, dstr0, dstr1, etr0, etr1,
              sidx0, sidx1, dsti0, dsti1, w0, w1, rows0, rows1,
              a_v, b_v, gw_v, node_v, den_v,
              acc_sp, den_sp, esem0, esem1, gsem0, gsem1, ssem0, ssem1):
    srcr = (srcr0, srcr1)
    dstr = (dstr0, dstr1)
    etr = (etr0, etr1)
    sidx = (sidx0, sidx1)
    dsti = (dsti0, dsti1)
    w_v = (w0, w1)
    rows_v = (rows0, rows1)
    esem = (esem0, esem1)
    gsem = (gsem0, gsem1)
    ssem = (ssem0, ssem1)
    c = lax.axis_index("c")
    s = lax.axis_index("s")
    lane = lax.iota(_i32, 16)
    z16 = jnp.zeros((16,), _f32)
    zi16 = jnp.zeros((16,), _i32)
    oi16 = jnp.ones((16,), _i32)

    def _zero_rows(i, _):
        node_v[i, pl.ds(0, 16)] = z16
        node_v[i, pl.ds(16, 16)] = z16
        node_v[i, pl.ds(32, 16)] = z16
        node_v[i, pl.ds(48, 16)] = z16
        return _
    lax.fori_loop(0, NB, _zero_rows, None)

    def _zero_den(i, _):
        den_v[pl.ds(i * 16, 16)] = z16
        return _
    lax.fori_loop(0, NPT // 16, _zero_den, None)

    for b in range(NPT // NB):
        pltpu.sync_copy(node_v, acc_sp.at[pl.ds(s * NPT + b * NB, NB)])
    pltpu.sync_copy(den_v, den_sp.at[pl.ds(s * NPT, NPT)])

    pltpu.sync_copy(scores.at[2 * c], a_v)
    pltpu.sync_copy(scores.at[2 * c + 1], b_v)
    pltpu.sync_copy(gw.at[c], gw_v)

    plsc.subcore_barrier()

    feat_base = c * NP
    edge_base = c * E + s * EPT

    def _edge_start(t, x):
        base = edge_base + t * K
        pltpu.async_copy(srcs.at[pl.ds(base, K)], srcr[x], esem[x])
        pltpu.async_copy(dsts.at[pl.ds(base, K)], dstr[x], esem[x])
        pltpu.async_copy(etype.at[pl.ds(base, K)], etr[x], esem[x])

    def _edge_wait(t, x):
        base = edge_base + t * K
        pltpu.make_async_copy(srcs.at[pl.ds(base, K)], srcr[x], esem[x]).wait()
        pltpu.make_async_copy(dsts.at[pl.ds(base, K)], dstr[x], esem[x]).wait()
        pltpu.make_async_copy(etype.at[pl.ds(base, K)], etr[x], esem[x]).wait()

    def _pass1(x):
        def _score(g, _c):
            j = g * 16
            src = srcr[x][pl.ds(j, 16)]
            dst = dstr[x][pl.ds(j, 16)]
            a_s = plsc.load_gather(a_v, [src])
            b_d = plsc.load_gather(b_v, [dst])
            et = etr[x][pl.ds(j, 16)]
            gb = plsc.load_gather(gw_v, [et])
            sc = a_s + b_d
            sc = jnp.where(sc >= 0.0, sc, 0.2 * sc) + gb
            w = 1.0 / (1.0 + jnp.exp(-sc))
            sidx[x][pl.ds(j, 16)] = src
            dsti[x][pl.ds(j, 16)] = dst + feat_base
            w_v[x][pl.ds(j, 16)] = w
            return _c
        lax.fori_loop(0, K // 16, _score, None)

    def _gather_start(x):
        pltpu.async_copy(feat.at[dsti[x]], rows_v[x], gsem[x])

    def _gather_wait(x):
        pltpu.make_async_copy(feat.at[dsti[x]], rows_v[x], gsem[x]).wait()

    def _scatter_start(x):
        pltpu.async_copy(rows_v[x], acc_sp.at[sidx[x]], ssem[x], add=True)
        pltpu.async_copy(w_v[x], den_sp.at[sidx[x]], ssem[x], add=True)

    def _scatter_wait(x):
        pltpu.make_async_copy(rows_v[x], acc_sp.at[sidx[x]], ssem[x]).wait()
        pltpu.make_async_copy(w_v[x], den_sp.at[sidx[x]], ssem[x]).wait()

    def _scale(x):
        @plsc.parallel_loop(0, K // 16)
        def _body(g):
            j = g * 16
            wv = w_v[x][pl.ds(j, 16)]
            for l in range(16):
                e = j + l
                w = wv[l]
                a0 = rows_v[x][e, pl.ds(0, 16)]
                a1 = rows_v[x][e, pl.ds(16, 16)]
                a2 = rows_v[x][e, pl.ds(32, 16)]
                a3 = rows_v[x][e, pl.ds(48, 16)]
                rows_v[x][e, pl.ds(0, 16)] = a0 * w
                rows_v[x][e, pl.ds(16, 16)] = a1 * w
                rows_v[x][e, pl.ds(32, 16)] = a2 * w
                rows_v[x][e, pl.ds(48, 16)] = a3 * w

    _edge_start(0, 0)
    _edge_wait(0, 0)
    _pass1(0)
    _gather_start(0)
    _edge_start(1, 1)
    _edge_start(2, 0)

    @pl.loop(0, NCHUNK, step=2)
    def _chunks(i):
        for b in (0, 1):
            t = i + b
            nb = 1 - b

            @pl.when(t + 1 < NCHUNK)
            def _():
                _edge_wait(t + 1, nb)

            _gather_wait(b)
            _scale(b)

            @pl.when(t >= 1)
            def _():
                _scatter_wait(nb)

            @pl.when(t + 1 < NCHUNK)
            def _():
                _pass1(nb)
                _gather_start(nb)

            @pl.when(t + 3 < NCHUNK)
            def _():
                _edge_start(t + 3, nb)

            _scatter_start(b)

    _scatter_wait((NCHUNK - 1) % 2)
    plsc.subcore_barrier()

    nbase = s * NPT
    pltpu.sync_copy(den_sp.at[pl.ds(nbase, NPT)], den_v)

    def _recip(g, _c):
        j = g * 16
        d = den_v[pl.ds(j, 16)]
        den_v[pl.ds(j, 16)] = 1.0 / jnp.maximum(d, 1e-8)
        return _c
    lax.fori_loop(0, NPT // 16, _recip, None)

    for b in range(NPT // NB):
        pltpu.sync_copy(acc_sp.at[pl.ds(nbase + b * NB, NB)], node_v)

        @plsc.parallel_loop(0, NB // 16)
        def _norm(g):
            j = g * 16
            rv = den_v[pl.ds(b * NB + j, 16)]
            for l in range(16):
                e = j + l
                r = rv[l]
                a0 = node_v[e, pl.ds(0, 16)]
                a1 = node_v[e, pl.ds(16, 16)]
                a2 = node_v[e, pl.ds(32, 16)]
                a3 = node_v[e, pl.ds(48, 16)]
                node_v[e, pl.ds(0, 16)] = a0 * r
                node_v[e, pl.ds(16, 16)] = a1 * r
                node_v[e, pl.ds(32, 16)] = a2 * r
                node_v[e, pl.ds(48, 16)] = a3 * r

        pltpu.sync_copy(node_v, out.at[pl.ds(feat_base + nbase + b * NB, NB)])


def _aggregate_sc(featf, scoresf, srcs, dsts, etype, gw2):
    mesh = plsc.VectorSubcoreMesh(core_axis_name="c", subcore_axis_name="s")
    return pl.kernel(
        _agg_body,
        out_type=jax.ShapeDtypeStruct((2 * NP, H), _f32),
        mesh=mesh,
        compiler_params=pltpu.CompilerParams(needs_layout_passes=False, use_tc_tiling_on_sc=False),
        scratch_types=[
            pltpu.VMEM((K,), _i32),
            pltpu.VMEM((K,), _i32),
            pltpu.VMEM((K,), _i32),
            pltpu.VMEM((K,), _i32),
            pltpu.VMEM((K,), _i32),
            pltpu.VMEM((K,), _i32),
            pltpu.VMEM((K,), _i32),
            pltpu.VMEM((K,), _i32),
            pltpu.VMEM((K,), _i32),
            pltpu.VMEM((K,), _i32),
            pltpu.VMEM((K,), _f32),
            pltpu.VMEM((K,), _f32),
            pltpu.VMEM((K, H), _f32),
            pltpu.VMEM((K, H), _f32),
            pltpu.VMEM((NP,), _f32),
            pltpu.VMEM((NP,), _f32),
            pltpu.VMEM((16,), _f32),
            pltpu.VMEM((NB, H), _f32),
            pltpu.VMEM((NPT,), _f32),
            pltpu.VMEM_SHARED((NP, H), _f32),
            pltpu.VMEM_SHARED((NP,), _f32),
            pltpu.SemaphoreType.DMA,
            pltpu.SemaphoreType.DMA,
            pltpu.SemaphoreType.DMA,
            pltpu.SemaphoreType.DMA,
            pltpu.SemaphoreType.DMA,
            pltpu.SemaphoreType.DMA,
        ],
    )(featf, scoresf, srcs, dsts, etype, gw2)


def _dense_out_body(agg_ref, x_ref, wdg_ref, wds_ref, b1_ref, wo_ref,
                    b2_ref, base_ref, emb_ref):
    a = agg_ref[0]
    x = x_ref[0]
    h = jnp.dot(a, wdg_ref[0], preferred_element_type=_f32)
    h = h + jnp.dot(x, wds_ref[0], preferred_element_type=_f32)
    h = jnp.maximum(h + b1_ref[0], 0.0)
    e = jnp.dot(h, wo_ref[0], preferred_element_type=_f32)
    e = jnp.maximum(e + b2_ref[0], 0.0)
    emb_ref[0] = e + base_ref[0]


def _dense_out(agg, xs, wdg, wds, b1, wo, b2, basep):
    return pl.pallas_call(
        _dense_out_body,
        grid=(2, NP // BM),
        in_specs=[
            pl.BlockSpec((1, BM, H), lambda g, m: (g, m, 0)),
            pl.BlockSpec((1, BM, F), lambda g, m: (g, m, 0)),
            pl.BlockSpec((1, H, H), lambda g, m: (g, 0, 0)),
            pl.BlockSpec((1, F, H), lambda g, m: (g, 0, 0)),
            pl.BlockSpec((1, 1, H), lambda g, m: (g, 0, 0)),
            pl.BlockSpec((1, H, H), lambda g, m: (g, 0, 0)),
            pl.BlockSpec((1, 1, H), lambda g, m: (g, 0, 0)),
            pl.BlockSpec((1, BM, H), lambda g, m: (g, m, 0)),
        ],
        out_specs=pl.BlockSpec((1, BM, H), lambda g, m: (g, m, 0)),
        out_shape=jax.ShapeDtypeStruct((2, NP, H), _f32),
    )(agg, xs, wdg, wds, b1, wo, b2, basep)


def _score_body(emb, uidx, bidx, ubias, bbias, gb, out,
                ui_v, bi_v, ue_v, be_v, ub_v, bb_v, bias_v, pred_v,
                sem, sem2, bsem):
    c = lax.axis_index("c")
    s = lax.axis_index("s")
    wid = s * 2 + c
    base = wid * BPW
    lane = lax.iota(_i32, 16)

    pltpu.sync_copy(uidx.at[pl.ds(base, BPW)], ui_v)
    pltpu.sync_copy(bidx.at[pl.ds(base, BPW)], bi_v)
    pltpu.async_copy(ubias, ub_v, bsem)
    pltpu.async_copy(bbias, bb_v, bsem)
    pltpu.sync_copy(gb, pred_v.at[pl.ds(0, 16)])
    gbv = pred_v[pl.ds(0, 16)]

    @plsc.parallel_loop(0, BPW // 16)
    def _off(g):
        j = g * 16
        bi_v[pl.ds(j, 16)] = bi_v[pl.ds(j, 16)] + NP

    pltpu.async_copy(emb.at[ui_v], ue_v, sem)
    pltpu.async_copy(emb.at[bi_v], be_v, sem2)

    pltpu.make_async_copy(ubias, ub_v, bsem).wait()
    pltpu.make_async_copy(bbias, bb_v, bsem).wait()

    def _prep(g, _c):
        j = g * 16
        ui = ui_v[pl.ds(j, 16)]
        bi = bi_v[pl.ds(j, 16)] - NP
        ub = plsc.load_gather(ub_v, [ui])
        bb = plsc.load_gather(bb_v, [bi])
        bias_v[pl.ds(j, 16)] = ub + bb + gbv
        return _c
    lax.fori_loop(0, BPW // 16, _prep, None)

    pltpu.make_async_copy(emb.at[ui_v], ue_v, sem).wait()
    pltpu.make_async_copy(emb.at[bi_v], be_v, sem2).wait()

    def _dot(g, _c):
        j = g * 16
        er = j + lane
        acc = jnp.zeros((16,), _f32)
        fc = jnp.zeros((16,), _i32)
        for _f in range(H):
            uv = plsc.load_gather(ue_v, [er, fc])
            bv = plsc.load_gather(be_v, [er, fc])
            acc = acc + uv * bv
            fc = fc + 1
        sc = acc + bias_v[pl.ds(j, 16)]
        pred = 4.0 / (1.0 + jnp.exp(-sc)) + 1.0
        pred_v[pl.ds(j, 16)] = pred
        return _c
    lax.fori_loop(0, BPW // 16, _dot, None)

    pltpu.sync_copy(pred_v, out.at[pl.ds(base, BPW)])


def _score_sc(embf, uidx, bidx, ubias, bbias, gb16):
    mesh = plsc.VectorSubcoreMesh(core_axis_name="c", subcore_axis_name="s")
    return pl.kernel(
        _score_body,
        out_type=jax.ShapeDtypeStruct((BATCH,), _f32),
        mesh=mesh,
        compiler_params=pltpu.CompilerParams(needs_layout_passes=False, use_tc_tiling_on_sc=False),
        scratch_types=[
            pltpu.VMEM((BPW,), _i32),
            pltpu.VMEM((BPW,), _i32),
            pltpu.VMEM((BPW, H), _f32),
            pltpu.VMEM((BPW, H), _f32),
            pltpu.VMEM((NP,), _f32),
            pltpu.VMEM((NP,), _f32),
            pltpu.VMEM((BPW,), _f32),
            pltpu.VMEM((BPW,), _f32),
            pltpu.SemaphoreType.DMA,
            pltpu.SemaphoreType.DMA,
            pltpu.SemaphoreType.DMA,
        ],
    )(embf, uidx, bidx, ubias, bbias, gb16)


def kernel(user_features, business_features, user_edges, business_edges,
           business_edge_type, user_idx, business_idx, Wu, Wb, att_u, att_b,
           graph_w, Wudg, budg, Wuds, buds, Wbdg, bbdg, Wbds, bbds, Wuo, buo,
           Wbo, bbo, user_base, business_base, user_bias_t, business_bias_t,
           global_bias):
    padn = ((0, NP - N_NODE), (0, 0))
    xs = jnp.stack([jnp.pad(user_features, padn),
                    jnp.pad(business_features, padn)])
    wt = jnp.stack([Wu.T, Wb.T])
    att2 = jnp.stack([jnp.stack([att_u[:H], att_u[H:]]),
                      jnp.stack([att_b[:H], att_b[H:]])])
    feat, scores = _dense_in(xs, wt, att2)

    srcs = jnp.concatenate([user_edges[:, 0], business_edges[:, 0]])
    dsts = jnp.concatenate([user_edges[:, 1], business_edges[:, 1]])
    etype = jnp.concatenate([jnp.zeros((E,), _i32),
                             business_edge_type], axis=0)
    gw2 = jnp.stack([jnp.zeros((16,), _f32),
                     jnp.pad(graph_w, (0, 13))])
    agg = _aggregate_sc(feat.reshape(2 * NP, H), scores.reshape(4, NP),
                        srcs, dsts, etype, gw2)

    b1 = jnp.stack([budg + buds, bbdg + bbds]).reshape(2, 1, H)
    b2 = jnp.stack([buo, bbo]).reshape(2, 1, H)
    wdg = jnp.stack([Wudg.T, Wbdg.T])
    wds = jnp.stack([Wuds.T, Wbds.T])
    wo = jnp.stack([Wuo.T, Wbo.T])
    basep = jnp.stack([jnp.pad(user_base, padn),
                       jnp.pad(business_base, padn)])
    emb = _dense_out(agg.reshape(2, NP, H), xs, wdg, wds, b1, wo, b2, basep)

    ubias = jnp.pad(user_bias_t[:, 0], (0, NP - N_NODE))
    bbias = jnp.pad(business_bias_t[:, 0], (0, NP - N_NODE))
    gb16 = jnp.full((16,), global_bias[0], _f32)
    pred = _score_sc(emb.reshape(2 * NP, H), user_idx, business_idx,
                     ubias, bbias, gb16)

    return (pred, emb[0, :N_NODE], emb[1, :N_NODE])


# speedup vs baseline: 1.1014x; 1.1014x over previous
import functools

import jax
import jax.numpy as jnp
from jax import lax
from jax.experimental import pallas as pl
from jax.experimental.pallas import tpu as pltpu
from jax.experimental.pallas import tpu_sc as plsc

N_NODE = 10000
H = 64
F = 128
E = 320000
BATCH = 16384

NT = 16
NP = 10240
EPT = E // NT
K = 400
NCHUNK = EPT // K
NPT = NP // NT
NB = 64
BM = 2560
BPW = BATCH // 32

_f32 = jnp.float32
_i32 = jnp.int32


def _dense_in_body(x_ref, w_ref, a2_ref, u1_ref, sc_ref):
    x = x_ref[0]
    w = w_ref[0]
    u1 = jnp.dot(x, w, preferred_element_type=_f32)
    u1_ref[0] = u1
    a2 = a2_ref[0]
    sc_ref[0] = lax.dot_general(a2, u1, (((1,), (1,)), ((), ())),
                                preferred_element_type=_f32)


def _dense_in(xs, wt, att2):
    return pl.pallas_call(
        _dense_in_body,
        grid=(2, NP // BM),
        in_specs=[
            pl.BlockSpec((1, BM, F), lambda g, m: (g, m, 0)),
            pl.BlockSpec((1, F, H), lambda g, m: (g, 0, 0)),
            pl.BlockSpec((1, 2, H), lambda g, m: (g, 0, 0)),
        ],
        out_specs=[
            pl.BlockSpec((1, BM, H), lambda g, m: (g, m, 0)),
            pl.BlockSpec((1, 2, BM), lambda g, m: (g, 0, m)),
        ],
        out_shape=[
            jax.ShapeDtypeStruct((2, NP, H), _f32),
            jax.ShapeDtypeStruct((2, 2, NP), _f32),
        ],
    )(xs, wt, att2)


def _agg_body(feat, scores, srcs, dsts, etype, gw, out,
              srcr0, srcr1, dstr0, dstr1, etr0, etr1,
              sidx0, sidx1, dsti0, dsti1, w0, w1, rows0, rows1,
              a_v, b_v, gw_v, node_v, den_v,
              acc_sp, den_sp, esem0, esem1, gsem0, gsem1, ssem0, ssem1):
    srcr = (srcr0, srcr1)
    dstr = (dstr0, dstr1)
    etr = (etr0, etr1)
    sidx = (sidx0, sidx1)
    dsti = (dsti0, dsti1)
    w_v = (w0, w1)
    rows_v = (rows0, rows1)
    esem = (esem0, esem1)
    gsem = (gsem0, gsem1)
    ssem = (ssem0, ssem1)
    c = lax.axis_index("c")
    s = lax.axis_index("s")
    lane = lax.iota(_i32, 16)
    z16 = jnp.zeros((16,), _f32)
    zi16 = jnp.zeros((16,), _i32)
    oi16 = jnp.ones((16,), _i32)

    def _zero_rows(i, _):
        node_v[i, pl.ds(0, 16)] = z16
        node_v[i, pl.ds(16, 16)] = z16
        node_v[i, pl.ds(32, 16)] = z16
        node_v[i, pl.ds(48, 16)] = z16
        return _
    lax.fori_loop(0, NB, _zero_rows, None)

    def _zero_den(i, _):
        den_v[pl.ds(i * 16, 16)] = z16
        return _
    lax.fori_loop(0, NPT // 16, _zero_den, None)

    for b in range(NPT // NB):
        pltpu.sync_copy(node_v, acc_sp.at[pl.ds(s * NPT + b * NB, NB)])
    pltpu.sync_copy(den_v, den_sp.at[pl.ds(s * NPT, NPT)])

    pltpu.sync_copy(scores.at[2 * c], a_v)
    pltpu.sync_copy(scores.at[2 * c + 1], b_v)
    pltpu.sync_copy(gw.at[c], gw_v)

    plsc.subcore_barrier()

    feat_base = c * NP
    edge_base = c * E + s * EPT

    def _edge_start(t, x):
        base = edge_base + t * K
        pltpu.async_copy(srcs.at[pl.ds(base, K)], srcr[x], esem[x])
        pltpu.async_copy(dsts.at[pl.ds(base, K)], dstr[x], esem[x])
        pltpu.async_copy(etype.at[pl.ds(base, K)], etr[x], esem[x])

    def _edge_wait(t, x):
        base = edge_base + t * K
        pltpu.make_async_copy(srcs.at[pl.ds(base, K)], srcr[x], esem[x]).wait()
        pltpu.make_async_copy(dsts.at[pl.ds(base, K)], dstr[x], esem[x]).wait()
        pltpu.make_async_copy(etype.at[pl.ds(base, K)], etr[x], esem[x]).wait()

    def _pass1(x):
        def _score(g, _c):
            j = g * 16
            src = srcr[x][pl.ds(j, 16)]
            dst = dstr[x][pl.ds(j, 16)]
            a_s = plsc.load_gather(a_v, [src])
            b_d = plsc.load_gather(b_v, [dst])
            et = etr[x][pl.ds(j, 16)]
            gb = plsc.load_gather(gw_v, [et])
            sc = a_s + b_d
            sc = jnp.where(sc >= 0.0, sc, 0.2 * sc) + gb
            w = 1.0 / (1.0 + jnp.exp(-sc))
            sidx[x][pl.ds(j, 16)] = src
            dsti[x][pl.ds(j, 16)] = dst + feat_base
            w_v[x][pl.ds(j, 16)] = w
            return _c
        lax.fori_loop(0, K // 16, _score, None)

    def _gather_start(x):
        pltpu.async_copy(feat.at[dsti[x]], rows_v[x], gsem[x])

    def _gather_wait(x):
        pltpu.make_async_copy(feat.at[dsti[x]], rows_v[x], gsem[x]).wait()

    def _scatter_start(x):
        pltpu.async_copy(rows_v[x], acc_sp.at[sidx[x]], ssem[x], add=True)
        pltpu.async_copy(w_v[x], den_sp.at[sidx[x]], ssem[x], add=True)

    def _scatter_wait(x):
        pltpu.make_async_copy(rows_v[x], acc_sp.at[sidx[x]], ssem[x]).wait()
        pltpu.make_async_copy(w_v[x], den_sp.at[sidx[x]], ssem[x]).wait()

    def _scale(x):
        @plsc.parallel_loop(0, K // 16)
        def _body(g):
            j = g * 16
            wv = w_v[x][pl.ds(j, 16)]
            for l in range(16):
                e = j + l
                w = wv[l]
                a0 = rows_v[x][e, pl.ds(0, 16)]
                a1 = rows_v[x][e, pl.ds(16, 16)]
                a2 = rows_v[x][e, pl.ds(32, 16)]
                a3 = rows_v[x][e, pl.ds(48, 16)]
                rows_v[x][e, pl.ds(0, 16)] = a0 * w
                rows_v[x][e, pl.ds(16, 16)] = a1 * w
                rows_v[x][e, pl.ds(32, 16)] = a2 * w
                rows_v[x][e, pl.ds(48, 16)] = a3 * w

    _edge_start(0, 0)
    _edge_wait(0, 0)
    _pass1(0)
    _gather_start(0)
    _edge_start(1, 1)
    _edge_start(2, 0)

    @pl.loop(0, NCHUNK, step=2)
    def _chunks(i):
        for b in (0, 1):
            t = i + b
            nb = 1 - b

            @pl.when(t >= 1)
            def _():
                _scatter_wait(nb)

            @pl.when(t + 1 < NCHUNK)
            def _():
                _edge_wait(t + 1, nb)
                _pass1(nb)
                _gather_start(nb)

            @pl.when(t + 3 < NCHUNK)
            def _():
                _edge_start(t + 3, nb)

            _gather_wait(b)
            _scale(b)
            _scatter_start(b)

    _scatter_wait((NCHUNK - 1) % 2)
    plsc.subcore_barrier()

    nbase = s * NPT
    pltpu.sync_copy(den_sp.at[pl.ds(nbase, NPT)], den_v)

    def _recip(g, _c):
        j = g * 16
        d = den_v[pl.ds(j, 16)]
        den_v[pl.ds(j, 16)] = 1.0 / jnp.maximum(d, 1e-8)
        return _c
    lax.fori_loop(0, NPT // 16, _recip, None)

    for b in range(NPT // NB):
        pltpu.sync_copy(acc_sp.at[pl.ds(nbase + b * NB, NB)], node_v)

        @plsc.parallel_loop(0, NB // 16)
        def _norm(g):
            j = g * 16
            rv = den_v[pl.ds(b * NB + j, 16)]
            for l in range(16):
                e = j + l
                r = rv[l]
                a0 = node_v[e, pl.ds(0, 16)]
                a1 = node_v[e, pl.ds(16, 16)]
                a2 = node_v[e, pl.ds(32, 16)]
                a3 = node_v[e, pl.ds(48, 16)]
                node_v[e, pl.ds(0, 16)] = a0 * r
                node_v[e, pl.ds(16, 16)] = a1 * r
                node_v[e, pl.ds(32, 16)] = a2 * r
                node_v[e, pl.ds(48, 16)] = a3 * r

        pltpu.sync_copy(node_v, out.at[pl.ds(feat_base + nbase + b * NB, NB)])


def _aggregate_sc(featf, scoresf, srcs, dsts, etype, gw2):
    mesh = plsc.VectorSubcoreMesh(core_axis_name="c", subcore_axis_name="s")
    return pl.kernel(
        _agg_body,
        out_type=jax.ShapeDtypeStruct((2 * NP, H), _f32),
        mesh=mesh,
        compiler_params=pltpu.CompilerParams(needs_layout_passes=False, use_tc_tiling_on_sc=False),
        scratch_types=[
            pltpu.VMEM((K,), _i32),
            pltpu.VMEM((K,), _i32),
            pltpu.VMEM((K,), _i32),
            pltpu.VMEM((K,), _i32),
            pltpu.VMEM((K,), _i32),
            pltpu.VMEM((K,), _i32),
            pltpu.VMEM((K,), _i32),
            pltpu.VMEM((K,), _i32),
            pltpu.VMEM((K,), _i32),
            pltpu.VMEM((K,), _i32),
            pltpu.VMEM((K,), _f32),
            pltpu.VMEM((K,), _f32),
            pltpu.VMEM((K, H), _f32),
            pltpu.VMEM((K, H), _f32),
            pltpu.VMEM((NP,), _f32),
            pltpu.VMEM((NP,), _f32),
            pltpu.VMEM((16,), _f32),
            pltpu.VMEM((NB, H), _f32),
            pltpu.VMEM((NPT,), _f32),
            pltpu.VMEM_SHARED((NP, H), _f32),
            pltpu.VMEM_SHARED((NP,), _f32),
            pltpu.SemaphoreType.DMA,
            pltpu.SemaphoreType.DMA,
            pltpu.SemaphoreType.DMA,
            pltpu.SemaphoreType.DMA,
            pltpu.SemaphoreType.DMA,
            pltpu.SemaphoreType.DMA,
        ],
    )(featf, scoresf, srcs, dsts, etype, gw2)


def _dense_out_body(agg_ref, x_ref, wdg_ref, wds_ref, b1_ref, wo_ref,
                    b2_ref, base_ref, emb_ref):
    a = agg_ref[0]
    x = x_ref[0]
    h = jnp.dot(a, wdg_ref[0], preferred_element_type=_f32)
    h = h + jnp.dot(x, wds_ref[0], preferred_element_type=_f32)
    h = jnp.maximum(h + b1_ref[0], 0.0)
    e = jnp.dot(h, wo_ref[0], preferred_element_type=_f32)
    e = jnp.maximum(e + b2_ref[0], 0.0)
    emb_ref[0] = e + base_ref[0]


def _dense_out(agg, xs, wdg, wds, b1, wo, b2, basep):
    return pl.pallas_call(
        _dense_out_body,
        grid=(2, NP // BM),
        in_specs=[
            pl.BlockSpec((1, BM, H), lambda g, m: (g, m, 0)),
            pl.BlockSpec((1, BM, F), lambda g, m: (g, m, 0)),
            pl.BlockSpec((1, H, H), lambda g, m: (g, 0, 0)),
            pl.BlockSpec((1, F, H), lambda g, m: (g, 0, 0)),
            pl.BlockSpec((1, 1, H), lambda g, m: (g, 0, 0)),
            pl.BlockSpec((1, H, H), lambda g, m: (g, 0, 0)),
            pl.BlockSpec((1, 1, H), lambda g, m: (g, 0, 0)),
            pl.BlockSpec((1, BM, H), lambda g, m: (g, m, 0)),
        ],
        out_specs=pl.BlockSpec((1, BM, H), lambda g, m: (g, m, 0)),
        out_shape=jax.ShapeDtypeStruct((2, NP, H), _f32),
    )(agg, xs, wdg, wds, b1, wo, b2, basep)


def _score_body(emb, uidx, bidx, ubias, bbias, gb, out,
                ui_v, bi_v, ue_v, be_v, ub_v, bb_v, bias_v, pred_v,
                sem, sem2, bsem):
    c = lax.axis_index("c")
    s = lax.axis_index("s")
    wid = s * 2 + c
    base = wid * BPW
    lane = lax.iota(_i32, 16)

    pltpu.sync_copy(uidx.at[pl.ds(base, BPW)], ui_v)
    pltpu.sync_copy(bidx.at[pl.ds(base, BPW)], bi_v)
    pltpu.async_copy(ubias, ub_v, bsem)
    pltpu.async_copy(bbias, bb_v, bsem)
    pltpu.sync_copy(gb, pred_v.at[pl.ds(0, 16)])
    gbv = pred_v[pl.ds(0, 16)]

    @plsc.parallel_loop(0, BPW // 16)
    def _off(g):
        j = g * 16
        bi_v[pl.ds(j, 16)] = bi_v[pl.ds(j, 16)] + NP

    pltpu.async_copy(emb.at[ui_v], ue_v, sem)
    pltpu.async_copy(emb.at[bi_v], be_v, sem2)

    pltpu.make_async_copy(ubias, ub_v, bsem).wait()
    pltpu.make_async_copy(bbias, bb_v, bsem).wait()

    def _prep(g, _c):
        j = g * 16
        ui = ui_v[pl.ds(j, 16)]
        bi = bi_v[pl.ds(j, 16)] - NP
        ub = plsc.load_gather(ub_v, [ui])
        bb = plsc.load_gather(bb_v, [bi])
        bias_v[pl.ds(j, 16)] = ub + bb + gbv
        return _c
    lax.fori_loop(0, BPW // 16, _prep, None)

    pltpu.make_async_copy(emb.at[ui_v], ue_v, sem).wait()
    pltpu.make_async_copy(emb.at[bi_v], be_v, sem2).wait()

    def _dot(g, _c):
        j = g * 16
        er = j + lane
        acc = jnp.zeros((16,), _f32)
        fc = jnp.zeros((16,), _i32)
        for _f in range(H):
            uv = plsc.load_gather(ue_v, [er, fc])
            bv = plsc.load_gather(be_v, [er, fc])
            acc = acc + uv * bv
            fc = fc + 1
        sc = acc + bias_v[pl.ds(j, 16)]
        pred = 4.0 / (1.0 + jnp.exp(-sc)) + 1.0
        pred_v[pl.ds(j, 16)] = pred
        return _c
    lax.fori_loop(0, BPW // 16, _dot, None)

    pltpu.sync_copy(pred_v, out.at[pl.ds(base, BPW)])


def _score_sc(embf, uidx, bidx, ubias, bbias, gb16):
    mesh = plsc.VectorSubcoreMesh(core_axis_name="c", subcore_axis_name="s")
    return pl.kernel(
        _score_body,
        out_type=jax.ShapeDtypeStruct((BATCH,), _f32),
        mesh=mesh,
        compiler_params=pltpu.CompilerParams(needs_layout_passes=False, use_tc_tiling_on_sc=False),
        scratch_types=[
            pltpu.VMEM((BPW,), _i32),
            pltpu.VMEM((BPW,), _i32),
            pltpu.VMEM((BPW, H), _f32),
            pltpu.VMEM((BPW, H), _f32),
            pltpu.VMEM((NP,), _f32),
            pltpu.VMEM((NP,), _f32),
            pltpu.VMEM((BPW,), _f32),
            pltpu.VMEM((BPW,), _f32),
            pltpu.SemaphoreType.DMA,
            pltpu.SemaphoreType.DMA,
            pltpu.SemaphoreType.DMA,
        ],
    )(embf, uidx, bidx, ubias, bbias, gb16)


def kernel(user_features, business_features, user_edges, business_edges,
           business_edge_type, user_idx, business_idx, Wu, Wb, att_u, att_b,
           graph_w, Wudg, budg, Wuds, buds, Wbdg, bbdg, Wbds, bbds, Wuo, buo,
           Wbo, bbo, user_base, business_base, user_bias_t, business_bias_t,
           global_bias):
    padn = ((0, NP - N_NODE), (0, 0))
    xs = jnp.stack([jnp.pad(user_features, padn),
                    jnp.pad(business_features, padn)])
    wt = jnp.stack([Wu.T, Wb.T])
    att2 = jnp.stack([jnp.stack([att_u[:H], att_u[H:]]),
                      jnp.stack([att_b[:H], att_b[H:]])])
    feat, scores = _dense_in(xs, wt, att2)

    srcs = jnp.concatenate([user_edges[:, 0], business_edges[:, 0]])
    dsts = jnp.concatenate([user_edges[:, 1], business_edges[:, 1]])
    etype = jnp.concatenate([jnp.zeros((E,), _i32),
                             business_edge_type], axis=0)
    gw2 = jnp.stack([jnp.zeros((16,), _f32),
                     jnp.pad(graph_w, (0, 13))])
    agg = _aggregate_sc(feat.reshape(2 * NP, H), scores.reshape(4, NP),
                        srcs, dsts, etype, gw2)

    b1 = jnp.stack([budg + buds, bbdg + bbds]).reshape(2, 1, H)
    b2 = jnp.stack([buo, bbo]).reshape(2, 1, H)
    wdg = jnp.stack([Wudg.T, Wbdg.T])
    wds = jnp.stack([Wuds.T, Wbds.T])
    wo = jnp.stack([Wuo.T, Wbo.T])
    basep = jnp.stack([jnp.pad(user_base, padn),
                       jnp.pad(business_base, padn)])
    emb = _dense_out(agg.reshape(2, NP, H), xs, wdg, wds, b1, wo, b2, basep)

    ubias = jnp.pad(user_bias_t[:, 0], (0, NP - N_NODE))
    bbias = jnp.pad(business_bias_t[:, 0], (0, NP - N_NODE))
    gb16 = jnp.full((16,), global_bias[0], _f32)
    pred = _score_sc(emb.reshape(2 * NP, H), user_idx, business_idx,
                     ubias, bbias, gb16)

    return (pred, emb[0, :N_NODE], emb[1, :N_NODE])
